# Initial kernel scaffold; baseline (speedup 1.0000x reference)
import functools

import numpy as np
import jax
import jax.numpy as jnp
from jax import lax
from jax.experimental import pallas as pl
from jax.experimental.pallas import tpu as pltpu

_N = 100000
_D = 128
_NUM_RIPPLES = 10
_SAMPLE = 10
_K = 1000
_SUB = 8
_LANES = _N // _SUB

_ripple_idx_cache = None


def _ripple_indices():
    global _ripple_idx_cache
    if _ripple_idx_cache is None:
        key = jax.random.key(42)
        size = _N // _NUM_RIPPLES
        parts = []
        for i in range(_NUM_RIPPLES):
            m = jax.random.permutation(jax.random.fold_in(key, i), size)[:_SAMPLE]
            parts.append(np.asarray(m, dtype=np.int64) + i * size)
        _ripple_idx_cache = tuple(int(v) for v in np.concatenate(parts))
    return _ripple_idx_cache


def _stage_a_body(idx, vel_ref, wp_ref, mp_ref, g_ref, be_ref, w1_ref, b1_ref,
                  w2_ref, b2_ref, out_ref):
    vel = vel_ref[...]
    wp = wp_ref[...]
    mp = mp_ref[...]

    norms = jnp.sum(vel * vel, axis=0)
    bits = lax.bitcast_convert_type(norms, jnp.int32)

    def search(i, t):
        cand = t | jnp.left_shift(jnp.int32(1), 30 - i)
        cnt = jnp.sum((bits >= cand).astype(jnp.int32))
        return jnp.where(cnt >= _K, cand, t)

    thresh = lax.fori_loop(0, 31, search, jnp.int32(0))
    mask = (bits >= thresh).astype(jnp.float32)

    hv_feat = [jnp.sum(vel[k] * mask) for k in range(3)]
    hv_wp = [jnp.sum(wp[k] * mask) for k in range(3)]
    hv_mp = [jnp.sum(mp[k] * mask) for k in range(2)]

    def gather(arr):
        cols = []
        for j in range(len(idx)):
            r, c = divmod(idx[j], _LANES)
            cols.append(arr[:, r:r + 1, c:c + 1])
        return jnp.concatenate(cols, axis=2)[:, 0, :]

    velg = gather(vel)
    wpg = gather(wp)
    mpg = gather(mp)

    rows = []
    for r in range(9):
        rows.append(jnp.broadcast_to(hv_feat[r % 3], (1, len(idx))))
    for k in range(3):
        rows.append(velg[k:k + 1, :])
    for k in range(3):
        rows.append(wpg[k:k + 1, :] - hv_wp[k])
    for k in range(2):
        rows.append(mpg[k:k + 1, :] - hv_mp[k])
    info = jnp.concatenate(rows, axis=0)

    mu = jnp.mean(info, axis=0, keepdims=True)
    var = jnp.mean((info - mu) ** 2, axis=0, keepdims=True)
    xn = (info - mu) * lax.rsqrt(var + 1e-5)
    xn = xn * g_ref[...] + be_ref[...]

    dn = (((0,), (0,)), ((), ()))
    h = lax.dot_general(w1_ref[...], xn, dn,
                        preferred_element_type=jnp.float32)
    h = jnp.maximum(h + b1_ref[...], 0.0)
    ot = lax.dot_general(w2_ref[...], h, dn,
                         preferred_element_type=jnp.float32)
    ot = ot + b2_ref[...]

    ii = lax.broadcasted_iota(jnp.int32, (_D, _D), 0)
    jj = lax.broadcasted_iota(jnp.int32, (_D, _D), 1)
    eye = (ii == jj).astype(jnp.float32)
    out_ref[...] = lax.dot_general(ot, eye, dn,
                                   preferred_element_type=jnp.float32)


def _stage_b_body(hits_by_block, in_ref, mlp_ref, out_ref):
    out_ref[...] = in_ref[...]
    pid = pl.program_id(0)
    for blk, hits in hits_by_block.items():
        @pl.when(pid == blk)
        def _(hits=hits):
            for row, j in hits:
                out_ref[pl.ds(row, 1), :] += mlp_ref[pl.ds(j, 1), :]


def kernel(latent_node_features, world_pos, mesh_pos, node_features,
           ln_gamma, ln_beta, W1, b1, W2, b2):
    idx = _ripple_indices()

    vel3 = node_features[:, :3].T.reshape(3, _SUB, _LANES)
    wp3 = world_pos.T.reshape(3, _SUB, _LANES)
    mp3 = mesh_pos.T.reshape(2, _SUB, _LANES)

    mlp_out = pl.pallas_call(
        functools.partial(_stage_a_body, idx),
        out_shape=jax.ShapeDtypeStruct((len(idx), _D), jnp.float32),
    )(vel3, wp3, mp3,
      ln_gamma.reshape(17, 1), ln_beta.reshape(17, 1),
      W1, b1.reshape(_D, 1), W2, b2.reshape(_D, 1))

    bs = 2000
    nblk = _N // bs
    hits_by_block = {}
    for j, t in enumerate(idx):
        hits_by_block.setdefault(t // bs, []).append((t % bs, j))

    out = pl.pallas_call(
        functools.partial(_stage_b_body, hits_by_block),
        grid=(nblk,),
        in_specs=[
            pl.BlockSpec((bs, _D), lambda i: (i, 0)),
            pl.BlockSpec((len(idx), _D), lambda i: (0, 0)),
        ],
        out_specs=pl.BlockSpec((bs, _D), lambda i: (i, 0)),
        out_shape=jax.ShapeDtypeStruct((_N, _D), jnp.float32),
    )(latent_node_features, mlp_out)
    return out


# speedup vs baseline: 14.4006x; 14.4006x over previous
import functools

import numpy as np
import jax
import jax.numpy as jnp
from jax import lax
from jax.experimental import pallas as pl
from jax.experimental.pallas import tpu as pltpu

_N = 100000
_D = 128
_NUM_RIPPLES = 10
_SAMPLE = 10
_K = 1000
_SUB = 8
_LANES = _N // _SUB

_ripple_idx_cache = None


def _ripple_indices():
    global _ripple_idx_cache
    if _ripple_idx_cache is None:
        with jax.ensure_compile_time_eval():
            key = jax.random.key(42)
            size = _N // _NUM_RIPPLES
            parts = []
            for i in range(_NUM_RIPPLES):
                m = jax.random.permutation(jax.random.fold_in(key, i), size)[:_SAMPLE]
                parts.append(np.asarray(m, dtype=np.int64) + i * size)
        _ripple_idx_cache = tuple(int(v) for v in np.concatenate(parts))
    return _ripple_idx_cache


def _stage_a_body(idx, vel_ref, wp_ref, mp_ref, g_ref, be_ref, w1_ref, b1_ref,
                  w2_ref, b2_ref, out_ref):
    vel = vel_ref[...]
    wp = wp_ref[...]
    mp = mp_ref[...]

    norms = jnp.sum(vel * vel, axis=0)
    bits = lax.bitcast_convert_type(norms, jnp.int32)

    def search(i, t):
        cand = t | jnp.left_shift(jnp.int32(1), 30 - i)
        cnt = jnp.sum((bits >= cand).astype(jnp.int32))
        return jnp.where(cnt >= _K, cand, t)

    thresh = lax.fori_loop(0, 31, search, jnp.int32(0))
    mask = (bits >= thresh).astype(jnp.float32)

    hv_feat = [jnp.sum(vel[k] * mask) for k in range(3)]
    hv_wp = [jnp.sum(wp[k] * mask) for k in range(3)]
    hv_mp = [jnp.sum(mp[k] * mask) for k in range(2)]

    def gather(arr):
        cols = []
        for j in range(len(idx)):
            r, c = divmod(idx[j], _LANES)
            cols.append(arr[:, r:r + 1, c:c + 1])
        return jnp.concatenate(cols, axis=2)[:, 0, :]

    velg = gather(vel)
    wpg = gather(wp)
    mpg = gather(mp)

    rows = []
    for r in range(9):
        rows.append(jnp.broadcast_to(hv_feat[r % 3], (1, len(idx))))
    for k in range(3):
        rows.append(velg[k:k + 1, :])
    for k in range(3):
        rows.append(wpg[k:k + 1, :] - hv_wp[k])
    for k in range(2):
        rows.append(mpg[k:k + 1, :] - hv_mp[k])
    info = jnp.concatenate(rows, axis=0)

    mu = jnp.mean(info, axis=0, keepdims=True)
    var = jnp.mean((info - mu) ** 2, axis=0, keepdims=True)
    xn = (info - mu) * lax.rsqrt(var + 1e-5)
    xn = xn * g_ref[...] + be_ref[...]

    dn = (((0,), (0,)), ((), ()))
    h = lax.dot_general(w1_ref[...], xn, dn,
                        preferred_element_type=jnp.float32)
    h = jnp.maximum(h + b1_ref[...], 0.0)
    ot = lax.dot_general(w2_ref[...], h, dn,
                         preferred_element_type=jnp.float32)
    ot = ot + b2_ref[...]

    ii = lax.broadcasted_iota(jnp.int32, (_D, _D), 0)
    jj = lax.broadcasted_iota(jnp.int32, (_D, _D), 1)
    eye = (ii == jj).astype(jnp.float32)
    out_ref[...] = lax.dot_general(ot, eye, dn,
                                   preferred_element_type=jnp.float32)


def _stage_b_body(hits_by_block, in_ref, mlp_ref, out_ref):
    out_ref[...] = in_ref[...]
    pid = pl.program_id(0)
    for blk, hits in hits_by_block.items():
        @pl.when(pid == blk)
        def _(hits=hits):
            for row, j in hits:
                out_ref[pl.ds(row, 1), :] += mlp_ref[pl.ds(j, 1), :]


def kernel(latent_node_features, world_pos, mesh_pos, node_features,
           ln_gamma, ln_beta, W1, b1, W2, b2):
    idx = _ripple_indices()

    vel3 = node_features[:, :3].T.reshape(3, _SUB, _LANES)
    wp3 = world_pos.T.reshape(3, _SUB, _LANES)
    mp3 = mesh_pos.T.reshape(2, _SUB, _LANES)

    mlp_out = pl.pallas_call(
        functools.partial(_stage_a_body, idx),
        out_shape=jax.ShapeDtypeStruct((len(idx), _D), jnp.float32),
    )(vel3, wp3, mp3,
      ln_gamma.reshape(17, 1), ln_beta.reshape(17, 1),
      W1, b1.reshape(_D, 1), W2, b2.reshape(_D, 1))

    bs = 2000
    nblk = _N // bs
    hits_by_block = {}
    for j, t in enumerate(idx):
        hits_by_block.setdefault(t // bs, []).append((t % bs, j))

    out = pl.pallas_call(
        functools.partial(_stage_b_body, hits_by_block),
        grid=(nblk,),
        in_specs=[
            pl.BlockSpec((bs, _D), lambda i: (i, 0)),
            pl.BlockSpec((len(idx), _D), lambda i: (0, 0)),
        ],
        out_specs=pl.BlockSpec((bs, _D), lambda i: (i, 0)),
        out_shape=jax.ShapeDtypeStruct((_N, _D), jnp.float32),
    )(latent_node_features, mlp_out)
    return out


# speedup vs baseline: 20.3780x; 1.4151x over previous
import functools

import numpy as np
import jax
import jax.numpy as jnp
from jax import lax
from jax.experimental import pallas as pl
from jax.experimental.pallas import tpu as pltpu

_N = 100000
_D = 128
_NUM_RIPPLES = 10
_SAMPLE = 10
_K = 1000
_SUB = 8
_LANES = _N // _SUB

_RIPPLE_IDX = (
    7931, 9798, 3642, 6342, 8569, 815, 8155, 6132, 1647, 5220,
    11695, 19558, 17770, 18095, 18949, 16390, 16501, 14647, 18316, 11753,
    24176, 20989, 24511, 26012, 23742, 24084, 23279, 23807, 28696, 22794,
    30259, 35283, 39620, 35632, 35902, 37603, 31260, 37474, 34929, 35963,
    43230, 44534, 41833, 41442, 49505, 45554, 47051, 47823, 40712, 45574,
    59314, 54153, 58245, 50512, 58903, 58725, 55572, 54341, 51034, 54554,
    64899, 62160, 62698, 61812, 68906, 62861, 69110, 67943, 62994, 61243,
    72174, 70609, 73374, 78334, 72275, 75048, 79856, 76859, 73444, 75207,
    84399, 82981, 84070, 89724, 84715, 80463, 81043, 87821, 89603, 81172,
    90501, 95295, 99142, 93197, 90414, 93168, 93949, 98508, 91139, 95091,
)


def _ripple_indices():
    return _RIPPLE_IDX


def _stage_a_body(idx, vel_ref, wp_ref, mp_ref, g_ref, be_ref, w1_ref, b1_ref,
                  w2_ref, b2_ref, out_ref):
    vel = vel_ref[...]
    wp = wp_ref[...]
    mp = mp_ref[...]

    norms = jnp.sum(vel * vel, axis=0)
    bits = lax.bitcast_convert_type(norms, jnp.int32)

    def search(i, t):
        cand = t | jnp.left_shift(jnp.int32(1), 30 - i)
        cnt = jnp.sum((bits >= cand).astype(jnp.int32))
        return jnp.where(cnt >= _K, cand, t)

    thresh = lax.fori_loop(0, 31, search, jnp.int32(0))
    mask = (bits >= thresh).astype(jnp.float32)

    hv_feat = [jnp.sum(vel[k] * mask) for k in range(3)]
    hv_wp = [jnp.sum(wp[k] * mask) for k in range(3)]
    hv_mp = [jnp.sum(mp[k] * mask) for k in range(2)]

    def gather(arr):
        cols = []
        for j in range(len(idx)):
            r, c = divmod(idx[j], _LANES)
            cols.append(arr[:, r:r + 1, c:c + 1])
        return jnp.concatenate(cols, axis=2)[:, 0, :]

    velg = gather(vel)
    wpg = gather(wp)
    mpg = gather(mp)

    rows = []
    for r in range(9):
        rows.append(jnp.broadcast_to(hv_feat[r % 3], (1, len(idx))))
    for k in range(3):
        rows.append(velg[k:k + 1, :])
    for k in range(3):
        rows.append(wpg[k:k + 1, :] - hv_wp[k])
    for k in range(2):
        rows.append(mpg[k:k + 1, :] - hv_mp[k])
    info = jnp.concatenate(rows, axis=0)

    mu = jnp.mean(info, axis=0, keepdims=True)
    var = jnp.mean((info - mu) ** 2, axis=0, keepdims=True)
    xn = (info - mu) * lax.rsqrt(var + 1e-5)
    xn = xn * g_ref[...] + be_ref[...]

    dn = (((0,), (0,)), ((), ()))
    h = lax.dot_general(w1_ref[...], xn, dn,
                        preferred_element_type=jnp.float32)
    h = jnp.maximum(h + b1_ref[...], 0.0)
    ot = lax.dot_general(w2_ref[...], h, dn,
                         preferred_element_type=jnp.float32)
    ot = ot + b2_ref[...]

    ii = lax.broadcasted_iota(jnp.int32, (_D, _D), 0)
    jj = lax.broadcasted_iota(jnp.int32, (_D, _D), 1)
    eye = (ii == jj).astype(jnp.float32)
    out_ref[...] = lax.dot_general(ot, eye, dn,
                                   preferred_element_type=jnp.float32)


def _stage_b_body(hits_by_block, in_ref, mlp_ref, out_ref):
    out_ref[...] = in_ref[...]
    pid = pl.program_id(0)
    for blk, hits in hits_by_block.items():
        @pl.when(pid == blk)
        def _(hits=hits):
            for row, j in hits:
                out_ref[pl.ds(row, 1), :] += mlp_ref[pl.ds(j, 1), :]


def kernel(latent_node_features, world_pos, mesh_pos, node_features,
           ln_gamma, ln_beta, W1, b1, W2, b2):
    idx = _ripple_indices()

    vel3 = node_features[:, :3].T.reshape(3, _SUB, _LANES)
    wp3 = world_pos.T.reshape(3, _SUB, _LANES)
    mp3 = mesh_pos.T.reshape(2, _SUB, _LANES)

    mlp_out = pl.pallas_call(
        functools.partial(_stage_a_body, idx),
        out_shape=jax.ShapeDtypeStruct((len(idx), _D), jnp.float32),
    )(vel3, wp3, mp3,
      ln_gamma.reshape(17, 1), ln_beta.reshape(17, 1),
      W1, b1.reshape(_D, 1), W2, b2.reshape(_D, 1))

    bs = 25000
    nblk = _N // bs
    hits_by_block = {}
    for j, t in enumerate(idx):
        hits_by_block.setdefault(t // bs, []).append((t % bs, j))

    out = pl.pallas_call(
        functools.partial(_stage_b_body, hits_by_block),
        grid=(nblk,),
        in_specs=[
            pl.BlockSpec((bs, _D), lambda i: (i, 0)),
            pl.BlockSpec((len(idx), _D), lambda i: (0, 0)),
        ],
        out_specs=pl.BlockSpec((bs, _D), lambda i: (i, 0)),
        out_shape=jax.ShapeDtypeStruct((_N, _D), jnp.float32),
    )(latent_node_features, mlp_out)
    return out
